# Initial kernel scaffold; baseline (speedup 1.0000x reference)
import jax
import jax.numpy as jnp
from jax.experimental import pallas as pl


def kernel(x, edge_index, w1a, b1a, w1b, b1b, g1, be1, w2a, b2a, w2b, b2b, g2, be2, w3a, b3a, w3b, b3b, g3, be3, w4a, b4a, w4b, b4b, g4, be4, w5a, b5a, w5b, b5b, g5, be5):
    raise NotImplementedError("write your pallas kernel here")


# speedup vs baseline: 2.0284x; 2.0284x over previous
import functools

import jax
import jax.numpy as jnp
import numpy as np
from jax import lax
from jax.experimental import pallas as pl
from jax.experimental.pallas import tpu as pltpu
from jax.experimental.pallas import tpu_sc as plsc

N = 10000
D = 128
E = 320000
NC = 2
NS = 16
NW = NC * NS
TROWS = 320
NP = NW * TROWS
K = 128
NCH = E // K
BAND = TROWS + 8


def _sc_agg_body(h_hbm, src_hbm, dst_hbm, bounds_hbm, zero_hbm, out_hbm,
                 bounds_v, src_v, dst_v, rows_v, band_v, sem):
    t = lax.axis_index("s") * NC + lax.axis_index("c")
    pltpu.sync_copy(zero_hbm, band_v)
    pltpu.sync_copy(bounds_hbm, bounds_v)
    b_lo = bounds_v[pl.ds(t, 16)][0]
    b_hi = bounds_v[pl.ds(t + 1, 16)][0]
    c_lo = lax.div(b_lo, K)
    c_hi = lax.div(b_hi + (K - 1), K)
    rowbase = t * TROWS

    def chunk_body(c, carry):
        off = c * K
        pltpu.sync_copy(src_hbm.at[pl.ds(off, K)], src_v)
        pltpu.sync_copy(dst_hbm.at[pl.ds(off, K)], dst_v.at[pl.ds(0, K)])
        pltpu.async_copy(h_hbm.at[src_v], rows_v, sem).wait()

        def edge_body(e, cc):
            s = dst_v[pl.ds(e, 16)][0] - rowbase
            s = jnp.where(jnp.logical_and(s >= 0, s < TROWS), s, TROWS)
            for q in range(8):
                plsc.addupdate(band_v.at[s, pl.ds(q * 16, 16)],
                               rows_v[e, pl.ds(q * 16, 16)])
            return cc

        lax.fori_loop(0, K, edge_body, 0)
        return carry

    lax.fori_loop(c_lo, c_hi, chunk_body, 0)
    pltpu.sync_copy(band_v.at[pl.ds(0, TROWS)],
                    out_hbm.at[pl.ds(rowbase, TROWS)])


@jax.jit
def _sc_agg(h, src_s, dst_s, bounds, zero_rows):
    mesh = plsc.VectorSubcoreMesh(core_axis_name="c", subcore_axis_name="s")
    return pl.kernel(
        _sc_agg_body,
        out_type=jax.ShapeDtypeStruct((NP, D), jnp.float32),
        mesh=mesh,
        scratch_types=[
            pltpu.VMEM((48,), jnp.int32),
            pltpu.VMEM((K,), jnp.int32),
            pltpu.VMEM((K + 16,), jnp.int32),
            pltpu.VMEM((K, D), jnp.float32),
            pltpu.VMEM((BAND, D), jnp.float32),
            pltpu.SemaphoreType.DMA,
        ],
    )(h, src_s, dst_s, bounds, zero_rows)


_INV_N = np.float32(1.0 / N)


def _fold8(acc):
    a = acc[:4] + acc[4:]
    b = a[:2] + a[2:]
    return b[:1] + b[1:]


def _tc_layer_body(h_ref, p_ref, wa_ref, ba_ref, wb_ref, bb_ref, g_ref,
                   be_ref, o_ref, hh_ref):
    hp = h_ref[...] + p_ref[pl.ds(0, N), :]
    z = jnp.maximum(lax.dot(hp, wa_ref[...]) + ba_ref[...], 0.0)
    hh_ref[...] = jnp.maximum(lax.dot(z, wb_ref[...]) + bb_ref[...], 0.0)

    def mstep(t, acc):
        return acc + hh_ref[pl.ds(t * 8, 8), :]

    m = _fold8(lax.fori_loop(0, N // 8, mstep, jnp.zeros((8, D), jnp.float32))
               ) * _INV_N

    def vstep(base):
        def step(t, acc):
            c = hh_ref[pl.ds(base + t * 8, 8), :] - m
            return acc + c * c
        return _fold8(lax.fori_loop(0, N // 16, step,
                                    jnp.zeros((8, D), jnp.float32)))

    v = (vstep(0) + vstep(N // 2)) * _INV_N
    o_ref[...] = (g_ref[...] * (hh_ref[...] - m)
                  * lax.rsqrt(v + 1e-5) + be_ref[...])


@jax.jit
def _tc_layer(h, p, wa, ba, wb, bb, g, be):
    return pl.pallas_call(
        _tc_layer_body,
        out_shape=jax.ShapeDtypeStruct((N, D), jnp.float32),
        scratch_shapes=[pltpu.VMEM((N, D), jnp.float32)],
    )(h, p, wa, ba, wb, bb, g, be)


def kernel(x, edge_index,
           w1a, b1a, w1b, b1b, g1, be1,
           w2a, b2a, w2b, b2b, g2, be2,
           w3a, b3a, w3b, b3b, g3, be3,
           w4a, b4a, w4b, b4b, g4, be4,
           w5a, b5a, w5b, b5b, g5, be5):
    dst_s, src_s = lax.sort((edge_index[1], edge_index[0]), num_keys=1,
                            is_stable=True)
    bounds = jnp.searchsorted(
        dst_s, jnp.arange(NW + 1, dtype=jnp.int32) * TROWS,
        side="left").astype(jnp.int32)
    bounds_p = jnp.zeros((48,), jnp.int32).at[:NW + 1].set(bounds)
    zero_rows = jnp.zeros((BAND, D), jnp.float32)

    w5b_p = jnp.zeros((D, D), jnp.float32).at[:, :2].set(w5b)
    b5b_p = jnp.zeros((D,), jnp.float32).at[:2].set(b5b)
    g5_p = jnp.zeros((D,), jnp.float32).at[:2].set(g5)
    be5_p = jnp.zeros((D,), jnp.float32).at[:2].set(be5)

    layers = [
        (w1a, b1a, w1b, b1b, g1, be1),
        (w2a, b2a, w2b, b2b, g2, be2),
        (w3a, b3a, w3b, b3b, g3, be3),
        (w4a, b4a, w4b, b4b, g4, be4),
        (w5a, b5a, w5b_p, b5b_p, g5_p, be5_p),
    ]

    h = x
    for (wa, ba, wb, bb, g, be) in layers:
        p = _sc_agg(h, src_s, dst_s, bounds_p, zero_rows)
        h = _tc_layer(h, p, wa, ba.reshape(1, D), wb, bb.reshape(1, D),
                      g.reshape(1, D), be.reshape(1, D))
    return h[:, :2]


# speedup vs baseline: 2.4228x; 1.1944x over previous
import functools

import jax
import jax.numpy as jnp
import numpy as np
from jax import lax
from jax.experimental import pallas as pl
from jax.experimental.pallas import tpu as pltpu
from jax.experimental.pallas import tpu_sc as plsc

N = 10000
D = 128
E = 320000
NC = 2
NS = 16
NW = NC * NS
TROWS = 320
NP = NW * TROWS
K = 128
NCH = E // K
BAND = TROWS + 8


def _sc_agg_body(h_hbm, src_hbm, dst_hbm, bounds_hbm, zero_hbm, out_hbm,
                 bounds_v, src_v, dst_v, rows_v, band_v, sem):
    t = lax.axis_index("s") * NC + lax.axis_index("c")
    pltpu.sync_copy(zero_hbm, band_v)
    pltpu.sync_copy(bounds_hbm, bounds_v)
    b_lo = bounds_v[pl.ds(t, 16)][0]
    b_hi = bounds_v[pl.ds(t + 1, 16)][0]
    c_lo = lax.div(b_lo, K)
    c_hi = lax.div(b_hi + (K - 1), K)
    rowbase = t * TROWS

    def chunk_body(c, carry):
        off = c * K
        pltpu.sync_copy(src_hbm.at[pl.ds(off, K)], src_v)
        pltpu.sync_copy(dst_hbm.at[pl.ds(off, K)], dst_v.at[pl.ds(0, K)])
        pltpu.async_copy(h_hbm.at[src_v], rows_v, sem).wait()

        def group_body(g, cc):
            base = g * 16
            dvec = dst_v[pl.ds(base, 16)] - rowbase
            for j in range(16):
                s = dvec[j]
                s = jnp.where(jnp.logical_and(s >= 0, s < TROWS), s, TROWS)
                for q in range(8):
                    plsc.addupdate(band_v.at[s, pl.ds(q * 16, 16)],
                                   rows_v[base + j, pl.ds(q * 16, 16)])
            return cc

        lax.fori_loop(0, K // 16, group_body, 0)
        return carry

    lax.fori_loop(c_lo, c_hi, chunk_body, 0)
    pltpu.sync_copy(band_v.at[pl.ds(0, TROWS)],
                    out_hbm.at[pl.ds(rowbase, TROWS)])


@jax.jit
def _sc_agg(h, src_s, dst_s, bounds, zero_rows):
    mesh = plsc.VectorSubcoreMesh(core_axis_name="c", subcore_axis_name="s")
    return pl.kernel(
        _sc_agg_body,
        out_type=jax.ShapeDtypeStruct((NP, D), jnp.float32),
        mesh=mesh,
        scratch_types=[
            pltpu.VMEM((48,), jnp.int32),
            pltpu.VMEM((K,), jnp.int32),
            pltpu.VMEM((K + 16,), jnp.int32),
            pltpu.VMEM((K, D), jnp.float32),
            pltpu.VMEM((BAND, D), jnp.float32),
            pltpu.SemaphoreType.DMA,
        ],
    )(h, src_s, dst_s, bounds, zero_rows)


_INV_N = np.float32(1.0 / N)


def _fold8(acc):
    a = acc[:4] + acc[4:]
    b = a[:2] + a[2:]
    return b[:1] + b[1:]


def _tc_layer_body(h_ref, p_ref, wa_ref, ba_ref, wb_ref, bb_ref, g_ref,
                   be_ref, o_ref, hh_ref):
    hp = h_ref[...] + p_ref[pl.ds(0, N), :]
    z = jnp.maximum(lax.dot(hp, wa_ref[...]) + ba_ref[...], 0.0)
    hh_ref[...] = jnp.maximum(lax.dot(z, wb_ref[...]) + bb_ref[...], 0.0)

    def mstep(t, acc):
        return acc + hh_ref[pl.ds(t * 8, 8), :]

    m = _fold8(lax.fori_loop(0, N // 8, mstep, jnp.zeros((8, D), jnp.float32))
               ) * _INV_N

    def vstep(base):
        def step(t, acc):
            c = hh_ref[pl.ds(base + t * 8, 8), :] - m
            return acc + c * c
        return _fold8(lax.fori_loop(0, N // 16, step,
                                    jnp.zeros((8, D), jnp.float32)))

    v = (vstep(0) + vstep(N // 2)) * _INV_N
    o_ref[...] = (g_ref[...] * (hh_ref[...] - m)
                  * lax.rsqrt(v + 1e-5) + be_ref[...])


@jax.jit
def _tc_layer(h, p, wa, ba, wb, bb, g, be):
    return pl.pallas_call(
        _tc_layer_body,
        out_shape=jax.ShapeDtypeStruct((N, D), jnp.float32),
        scratch_shapes=[pltpu.VMEM((N, D), jnp.float32)],
    )(h, p, wa, ba, wb, bb, g, be)


def kernel(x, edge_index,
           w1a, b1a, w1b, b1b, g1, be1,
           w2a, b2a, w2b, b2b, g2, be2,
           w3a, b3a, w3b, b3b, g3, be3,
           w4a, b4a, w4b, b4b, g4, be4,
           w5a, b5a, w5b, b5b, g5, be5):
    dst_s, src_s = lax.sort((edge_index[1], edge_index[0]), num_keys=1,
                            is_stable=True)
    bounds = jnp.searchsorted(
        dst_s, jnp.arange(NW + 1, dtype=jnp.int32) * TROWS,
        side="left").astype(jnp.int32)
    bounds_p = jnp.zeros((48,), jnp.int32).at[:NW + 1].set(bounds)
    zero_rows = jnp.zeros((BAND, D), jnp.float32)

    w5b_p = jnp.zeros((D, D), jnp.float32).at[:, :2].set(w5b)
    b5b_p = jnp.zeros((D,), jnp.float32).at[:2].set(b5b)
    g5_p = jnp.zeros((D,), jnp.float32).at[:2].set(g5)
    be5_p = jnp.zeros((D,), jnp.float32).at[:2].set(be5)

    layers = [
        (w1a, b1a, w1b, b1b, g1, be1),
        (w2a, b2a, w2b, b2b, g2, be2),
        (w3a, b3a, w3b, b3b, g3, be3),
        (w4a, b4a, w4b, b4b, g4, be4),
        (w5a, b5a, w5b_p, b5b_p, g5_p, be5_p),
    ]

    h = x
    for (wa, ba, wb, bb, g, be) in layers:
        p = _sc_agg(h, src_s, dst_s, bounds_p, zero_rows)
        h = _tc_layer(h, p, wa, ba.reshape(1, D), wb, bb.reshape(1, D),
                      g.reshape(1, D), be.reshape(1, D))
    return h[:, :2]


# speedup vs baseline: 3.4988x; 1.4441x over previous
import functools

import jax
import jax.numpy as jnp
import numpy as np
from jax import lax
from jax.experimental import pallas as pl
from jax.experimental.pallas import tpu as pltpu
from jax.experimental.pallas import tpu_sc as plsc

N = 10000
D = 128
E = 320000
NC = 2
NS = 16
NW = NC * NS
TROWS = 320
NP = NW * TROWS
K = 128
NCH = E // K
BAND = TROWS + 8


def _sc_agg_body(h_hbm, src_hbm, dst_hbm, bounds_hbm, zero_hbm, out_hbm,
                 bounds_v, src_v, dst_v, rows_v, band_v, sem):
    t = lax.axis_index("s") * NC + lax.axis_index("c")
    pltpu.sync_copy(zero_hbm, band_v)
    pltpu.sync_copy(bounds_hbm, bounds_v)
    b_lo = bounds_v[pl.ds(t, 16)][0]
    b_hi = bounds_v[pl.ds(t + 1, 16)][0]
    c_lo = lax.div(b_lo, K)
    c_hi = lax.div(b_hi + (K - 1), K)
    rowbase = t * TROWS

    zero16 = jnp.zeros((16,), jnp.float32)

    def flush(s_prev, acc):
        for q in range(8):
            plsc.addupdate(band_v.at[pl.ds(s_prev * D + q * 16, 16)], acc[q])

    def chunk_body(c, st):
        off = c * K
        pltpu.sync_copy(src_hbm.at[pl.ds(off, K)], src_v)
        pltpu.sync_copy(dst_hbm.at[pl.ds(off, K)], dst_v.at[pl.ds(0, K)])
        pltpu.async_copy(h_hbm.at[src_v], rows_v, sem).wait()

        def group_body(g, st2):
            s_prev, acc = st2
            base = g * 16
            dvec = dst_v[pl.ds(base, 16)] - rowbase
            for j in range(16):
                s = dvec[j]
                s = jnp.where(jnp.logical_and(s >= 0, s < TROWS), s, TROWS)
                cond = s != s_prev

                @pl.when(cond)
                def _(s_prev=s_prev, acc=acc):
                    flush(s_prev, acc)

                r = [rows_v[base + j, pl.ds(q * 16, 16)] for q in range(8)]
                acc = tuple(
                    jnp.where(cond, r[q], acc[q] + r[q]) for q in range(8))
                s_prev = s
            return (s_prev, acc)

        return lax.fori_loop(0, K // 16, group_body, st)

    st = lax.fori_loop(c_lo, c_hi, chunk_body,
                       (jnp.int32(TROWS), (zero16,) * 8))
    flush(st[0], st[1])
    pltpu.sync_copy(band_v.at[pl.ds(0, TROWS * D)],
                    out_hbm.at[pl.ds(rowbase * D, TROWS * D)])


@jax.jit
def _sc_agg(h, src_s, dst_s, bounds, zero_rows):
    mesh = plsc.VectorSubcoreMesh(core_axis_name="c", subcore_axis_name="s")
    return pl.kernel(
        _sc_agg_body,
        out_type=jax.ShapeDtypeStruct((NP * D,), jnp.float32),
        mesh=mesh,
        scratch_types=[
            pltpu.VMEM((48,), jnp.int32),
            pltpu.VMEM((K,), jnp.int32),
            pltpu.VMEM((K + 16,), jnp.int32),
            pltpu.VMEM((K, D), jnp.float32),
            pltpu.VMEM((BAND * D,), jnp.float32),
            pltpu.SemaphoreType.DMA,
        ],
    )(h, src_s, dst_s, bounds, zero_rows)


_INV_N = np.float32(1.0 / N)


def _fold8(acc):
    a = acc[:4] + acc[4:]
    b = a[:2] + a[2:]
    return b[:1] + b[1:]


def _tc_layer_body(h_ref, p_ref, wa_ref, ba_ref, wb_ref, bb_ref, g_ref,
                   be_ref, o_ref, hh_ref):
    hp = h_ref[...] + p_ref[pl.ds(0, N), :]
    z = jnp.maximum(lax.dot(hp, wa_ref[...]) + ba_ref[...], 0.0)
    hh_ref[...] = jnp.maximum(lax.dot(z, wb_ref[...]) + bb_ref[...], 0.0)

    def mstep(t, acc):
        return acc + hh_ref[pl.ds(t * 8, 8), :]

    m = _fold8(lax.fori_loop(0, N // 8, mstep, jnp.zeros((8, D), jnp.float32))
               ) * _INV_N

    def vstep(base):
        def step(t, acc):
            c = hh_ref[pl.ds(base + t * 8, 8), :] - m
            return acc + c * c
        return _fold8(lax.fori_loop(0, N // 16, step,
                                    jnp.zeros((8, D), jnp.float32)))

    v = (vstep(0) + vstep(N // 2)) * _INV_N
    o_ref[...] = (g_ref[...] * (hh_ref[...] - m)
                  * lax.rsqrt(v + 1e-5) + be_ref[...])


@jax.jit
def _tc_layer(h, p, wa, ba, wb, bb, g, be):
    return pl.pallas_call(
        _tc_layer_body,
        out_shape=jax.ShapeDtypeStruct((N, D), jnp.float32),
        scratch_shapes=[pltpu.VMEM((N, D), jnp.float32)],
    )(h, p, wa, ba, wb, bb, g, be)


def kernel(x, edge_index,
           w1a, b1a, w1b, b1b, g1, be1,
           w2a, b2a, w2b, b2b, g2, be2,
           w3a, b3a, w3b, b3b, g3, be3,
           w4a, b4a, w4b, b4b, g4, be4,
           w5a, b5a, w5b, b5b, g5, be5):
    dst_s, src_s = lax.sort((edge_index[1], edge_index[0]), num_keys=1,
                            is_stable=True)
    bounds = jnp.searchsorted(
        dst_s, jnp.arange(NW + 1, dtype=jnp.int32) * TROWS,
        side="left").astype(jnp.int32)
    bounds_p = jnp.zeros((48,), jnp.int32).at[:NW + 1].set(bounds)
    zero_rows = jnp.zeros((BAND * D,), jnp.float32)

    w5b_p = jnp.zeros((D, D), jnp.float32).at[:, :2].set(w5b)
    b5b_p = jnp.zeros((D,), jnp.float32).at[:2].set(b5b)
    g5_p = jnp.zeros((D,), jnp.float32).at[:2].set(g5)
    be5_p = jnp.zeros((D,), jnp.float32).at[:2].set(be5)

    layers = [
        (w1a, b1a, w1b, b1b, g1, be1),
        (w2a, b2a, w2b, b2b, g2, be2),
        (w3a, b3a, w3b, b3b, g3, be3),
        (w4a, b4a, w4b, b4b, g4, be4),
        (w5a, b5a, w5b_p, b5b_p, g5_p, be5_p),
    ]

    h = x
    for (wa, ba, wb, bb, g, be) in layers:
        p = _sc_agg(h, src_s, dst_s, bounds_p, zero_rows).reshape(NP, D)
        h = _tc_layer(h, p, wa, ba.reshape(1, D), wb, bb.reshape(1, D),
                      g.reshape(1, D), be.reshape(1, D))
    return h[:, :2]


# speedup vs baseline: 5.0236x; 1.4358x over previous
import functools

import jax
import jax.numpy as jnp
import numpy as np
from jax import lax
from jax.experimental import pallas as pl
from jax.experimental.pallas import tpu as pltpu
from jax.experimental.pallas import tpu_sc as plsc

N = 10000
D = 128
E = 320000
NC = 2
NS = 16
NW = NC * NS
TROWS = 320
NP = NW * TROWS
K = 128
NCHP = E // K + 1
EP = NCHP * K
BAND = TROWS + 8


def _sc_agg_body(h_hbm, src_hbm, dst_hbm, bounds_hbm, zero_hbm, out_hbm,
                 bounds_v, src_v0, dst_v0, src_v1, dst_v1, rows_v0, rows_v1,
                 band_v, semg0, semg1, semi0, semi1):
    t = lax.axis_index("s") * NC + lax.axis_index("c")
    pltpu.sync_copy(zero_hbm, band_v)
    pltpu.sync_copy(bounds_hbm, bounds_v)
    b_lo = bounds_v[pl.ds(t, 16)][0]
    b_hi = bounds_v[pl.ds(t + 1, 16)][0]
    c_lo = lax.div(b_lo, K)
    c_hi = lax.div(b_hi + (K - 1), K)
    npairs = lax.max((c_hi - c_lo + 1) // 2, 1)
    rowbase = t * TROWS

    zero16 = jnp.zeros((16,), jnp.float32)

    def flush(s_prev, acc):
        for q in range(8):
            plsc.addupdate(band_v.at[pl.ds(s_prev * D + q * 16, 16)], acc[q])

    def start_idx(c, src_v, dst_v, semi):
        off = jnp.minimum(c, NCHP - 1) * K
        pltpu.async_copy(src_hbm.at[pl.ds(off, K)], src_v, semi)
        pltpu.async_copy(dst_hbm.at[pl.ds(off, K)], dst_v.at[pl.ds(0, K)],
                         semi)

    def wait_idx(src_v, dst_v, semi):
        pltpu.make_async_copy(src_hbm.at[pl.ds(0, K)], src_v, semi).wait()
        pltpu.make_async_copy(dst_hbm.at[pl.ds(0, K)],
                              dst_v.at[pl.ds(0, K)], semi).wait()

    def start_gather(src_v, rows_v, semg):
        pltpu.async_copy(h_hbm.at[src_v], rows_v, semg)

    def wait_gather(src_v, rows_v, semg):
        pltpu.make_async_copy(h_hbm.at[src_v], rows_v, semg).wait()

    def process(dst_v, rows_v, st):
        def group_body(g, st2):
            s_prev, acc = st2
            base = g * 16
            dvec = dst_v[pl.ds(base, 16)] - rowbase
            for j in range(16):
                s = dvec[j]
                s = jnp.where(jnp.logical_and(s >= 0, s < TROWS), s, TROWS)
                cond = s != s_prev

                @pl.when(cond)
                def _(s_prev=s_prev, acc=acc):
                    flush(s_prev, acc)

                r = [rows_v[base + j, pl.ds(q * 16, 16)] for q in range(8)]
                acc = tuple(
                    jnp.where(cond, r[q], acc[q] + r[q]) for q in range(8))
                s_prev = s
            return (s_prev, acc)

        return lax.fori_loop(0, K // 16, group_body, st)

    start_idx(c_lo, src_v0, dst_v0, semi0)
    wait_idx(src_v0, dst_v0, semi0)
    start_gather(src_v0, rows_v0, semg0)
    start_idx(c_lo + 1, src_v1, dst_v1, semi1)

    def pair_body(pi, st):
        a = c_lo + 2 * pi
        wait_idx(src_v1, dst_v1, semi1)
        start_gather(src_v1, rows_v1, semg1)
        wait_gather(src_v0, rows_v0, semg0)
        st = process(dst_v0, rows_v0, st)
        start_idx(a + 2, src_v0, dst_v0, semi0)
        wait_gather(src_v1, rows_v1, semg1)
        st = process(dst_v1, rows_v1, st)
        wait_idx(src_v0, dst_v0, semi0)
        start_gather(src_v0, rows_v0, semg0)
        start_idx(a + 3, src_v1, dst_v1, semi1)
        return st

    st = lax.fori_loop(0, npairs, pair_body,
                       (jnp.int32(TROWS), (zero16,) * 8))
    wait_gather(src_v0, rows_v0, semg0)
    wait_idx(src_v1, dst_v1, semi1)
    flush(st[0], st[1])
    pltpu.sync_copy(band_v.at[pl.ds(0, TROWS * D)],
                    out_hbm.at[pl.ds(rowbase * D, TROWS * D)])


@jax.jit
def _sc_agg(h, src_s, dst_s, bounds, zero_rows):
    mesh = plsc.VectorSubcoreMesh(core_axis_name="c", subcore_axis_name="s")
    return pl.kernel(
        _sc_agg_body,
        out_type=jax.ShapeDtypeStruct((NP * D,), jnp.float32),
        mesh=mesh,
        scratch_types=[
            pltpu.VMEM((48,), jnp.int32),
            pltpu.VMEM((K,), jnp.int32),
            pltpu.VMEM((K + 16,), jnp.int32),
            pltpu.VMEM((K,), jnp.int32),
            pltpu.VMEM((K + 16,), jnp.int32),
            pltpu.VMEM((K, D), jnp.float32),
            pltpu.VMEM((K, D), jnp.float32),
            pltpu.VMEM((BAND * D,), jnp.float32),
            pltpu.SemaphoreType.DMA,
            pltpu.SemaphoreType.DMA,
            pltpu.SemaphoreType.DMA,
            pltpu.SemaphoreType.DMA,
        ],
    )(h, src_s, dst_s, bounds, zero_rows)


_INV_N = np.float32(1.0 / N)


def _fold8(acc):
    a = acc[:4] + acc[4:]
    b = a[:2] + a[2:]
    return b[:1] + b[1:]


def _tc_layer_body(h_ref, p_ref, wa_ref, ba_ref, wb_ref, bb_ref, g_ref,
                   be_ref, o_ref, hh_ref):
    hp = h_ref[...] + p_ref[pl.ds(0, N), :]
    z = jnp.maximum(lax.dot(hp, wa_ref[...]) + ba_ref[...], 0.0)
    hh_ref[...] = jnp.maximum(lax.dot(z, wb_ref[...]) + bb_ref[...], 0.0)

    def mstep(t, acc):
        return acc + hh_ref[pl.ds(t * 8, 8), :]

    m = _fold8(lax.fori_loop(0, N // 8, mstep, jnp.zeros((8, D), jnp.float32))
               ) * _INV_N

    def vstep(base):
        def step(t, acc):
            c = hh_ref[pl.ds(base + t * 8, 8), :] - m
            return acc + c * c
        return _fold8(lax.fori_loop(0, N // 16, step,
                                    jnp.zeros((8, D), jnp.float32)))

    v = (vstep(0) + vstep(N // 2)) * _INV_N
    o_ref[...] = (g_ref[...] * (hh_ref[...] - m)
                  * lax.rsqrt(v + 1e-5) + be_ref[...])


@jax.jit
def _tc_layer(h, p, wa, ba, wb, bb, g, be):
    return pl.pallas_call(
        _tc_layer_body,
        out_shape=jax.ShapeDtypeStruct((N, D), jnp.float32),
        scratch_shapes=[pltpu.VMEM((N, D), jnp.float32)],
    )(h, p, wa, ba, wb, bb, g, be)


def kernel(x, edge_index,
           w1a, b1a, w1b, b1b, g1, be1,
           w2a, b2a, w2b, b2b, g2, be2,
           w3a, b3a, w3b, b3b, g3, be3,
           w4a, b4a, w4b, b4b, g4, be4,
           w5a, b5a, w5b, b5b, g5, be5):
    dst_s, src_s = lax.sort((edge_index[1], edge_index[0]), num_keys=1,
                            is_stable=True)
    bounds = jnp.searchsorted(
        dst_s, jnp.arange(NW + 1, dtype=jnp.int32) * TROWS,
        side="left").astype(jnp.int32)
    src_s = jnp.concatenate([src_s, jnp.zeros((K,), jnp.int32)])
    dst_s = jnp.concatenate([dst_s, jnp.full((K,), NP, jnp.int32)])
    bounds_p = jnp.zeros((48,), jnp.int32).at[:NW + 1].set(bounds)
    zero_rows = jnp.zeros((BAND * D,), jnp.float32)

    w5b_p = jnp.zeros((D, D), jnp.float32).at[:, :2].set(w5b)
    b5b_p = jnp.zeros((D,), jnp.float32).at[:2].set(b5b)
    g5_p = jnp.zeros((D,), jnp.float32).at[:2].set(g5)
    be5_p = jnp.zeros((D,), jnp.float32).at[:2].set(be5)

    layers = [
        (w1a, b1a, w1b, b1b, g1, be1),
        (w2a, b2a, w2b, b2b, g2, be2),
        (w3a, b3a, w3b, b3b, g3, be3),
        (w4a, b4a, w4b, b4b, g4, be4),
        (w5a, b5a, w5b_p, b5b_p, g5_p, be5_p),
    ]

    h = x
    for (wa, ba, wb, bb, g, be) in layers:
        p = _sc_agg(h, src_s, dst_s, bounds_p, zero_rows).reshape(NP, D)
        h = _tc_layer(h, p, wa, ba.reshape(1, D), wb, bb.reshape(1, D),
                      g.reshape(1, D), be.reshape(1, D))
    return h[:, :2]


# speedup vs baseline: 5.2025x; 1.0356x over previous
import functools

import jax
import jax.numpy as jnp
import numpy as np
from jax import lax
from jax.experimental import pallas as pl
from jax.experimental.pallas import tpu as pltpu
from jax.experimental.pallas import tpu_sc as plsc

N = 10000
D = 128
E = 320000
NC = 2
NS = 16
NW = NC * NS
TROWS = 320
NP = NW * TROWS
K = 128
NCHP = E // K + 1
EP = NCHP * K
BAND = TROWS + 8


def _sc_agg_body(h_hbm, src_hbm, dst_hbm, bounds_hbm, zero_hbm, out_hbm,
                 bounds_v, src_v0, dst_v0, src_v1, dst_v1, rows_v0, rows_v1,
                 band_v, semg0, semg1, semi0, semi1):
    t = lax.axis_index("s") * NC + lax.axis_index("c")
    pltpu.sync_copy(zero_hbm, band_v)
    pltpu.sync_copy(bounds_hbm, bounds_v)
    b_lo = bounds_v[pl.ds(t, 16)][0]
    b_hi = bounds_v[pl.ds(t + 1, 16)][0]
    c_lo = lax.div(b_lo, K)
    c_hi = lax.div(b_hi + (K - 1), K)
    npairs = lax.max((c_hi - c_lo + 1) // 2, 1)
    rowbase = t * TROWS

    zero16 = jnp.zeros((16,), jnp.float32)

    def flush(s_prev, acc):
        for q in range(8):
            plsc.addupdate(band_v.at[pl.ds(s_prev * D + q * 16, 16)], acc[q])

    def start_idx(c, src_v, dst_v, semi):
        off = jnp.minimum(c, NCHP - 1) * K
        pltpu.async_copy(src_hbm.at[pl.ds(off, K)], src_v, semi)
        pltpu.async_copy(dst_hbm.at[pl.ds(off, K)], dst_v.at[pl.ds(0, K)],
                         semi)

    def wait_idx(src_v, dst_v, semi):
        pltpu.make_async_copy(src_hbm.at[pl.ds(0, K)], src_v, semi).wait()
        pltpu.make_async_copy(dst_hbm.at[pl.ds(0, K)],
                              dst_v.at[pl.ds(0, K)], semi).wait()

    def start_gather(src_v, rows_v, semg):
        pltpu.async_copy(h_hbm.at[src_v], rows_v, semg)

    def wait_gather(src_v, rows_v, semg):
        pltpu.make_async_copy(h_hbm.at[src_v], rows_v, semg).wait()

    def process(dst_v, rows_v, st):
        def group_body(g, st2):
            s_prev, acc = st2
            base = g * 16
            dvec = dst_v[pl.ds(base, 16)] - rowbase
            for j in range(16):
                s = dvec[j]
                s = jnp.where(jnp.logical_and(s >= 0, s < TROWS), s, TROWS)
                cond = s != s_prev

                @pl.when(cond)
                def _(s_prev=s_prev, acc=acc):
                    flush(s_prev, acc)

                r = [rows_v[base + j, pl.ds(q * 16, 16)] for q in range(8)]
                acc = tuple(
                    jnp.where(cond, r[q], acc[q] + r[q]) for q in range(8))
                s_prev = s
            return (s_prev, acc)

        return lax.fori_loop(0, K // 16, group_body, st)

    start_idx(c_lo, src_v0, dst_v0, semi0)
    wait_idx(src_v0, dst_v0, semi0)
    start_gather(src_v0, rows_v0, semg0)
    start_idx(c_lo + 1, src_v1, dst_v1, semi1)

    def pair_body(pi, st):
        a = c_lo + 2 * pi
        wait_idx(src_v1, dst_v1, semi1)
        start_gather(src_v1, rows_v1, semg1)
        wait_gather(src_v0, rows_v0, semg0)
        st = process(dst_v0, rows_v0, st)
        start_idx(a + 2, src_v0, dst_v0, semi0)
        wait_gather(src_v1, rows_v1, semg1)
        st = process(dst_v1, rows_v1, st)
        wait_idx(src_v0, dst_v0, semi0)
        start_gather(src_v0, rows_v0, semg0)
        start_idx(a + 3, src_v1, dst_v1, semi1)
        return st

    st = lax.fori_loop(0, npairs, pair_body,
                       (jnp.int32(TROWS), (zero16,) * 8))
    wait_gather(src_v0, rows_v0, semg0)
    wait_idx(src_v1, dst_v1, semi1)
    flush(st[0], st[1])
    pltpu.sync_copy(band_v.at[pl.ds(0, TROWS * D)],
                    out_hbm.at[pl.ds(rowbase * D, TROWS * D)])


@jax.jit
def _sc_agg(h, src_s, dst_s, bounds, zero_rows):
    mesh = plsc.VectorSubcoreMesh(core_axis_name="c", subcore_axis_name="s")
    return pl.kernel(
        _sc_agg_body,
        out_type=jax.ShapeDtypeStruct((NP * D,), jnp.float32),
        mesh=mesh,
        scratch_types=[
            pltpu.VMEM((48,), jnp.int32),
            pltpu.VMEM((K,), jnp.int32),
            pltpu.VMEM((K + 16,), jnp.int32),
            pltpu.VMEM((K,), jnp.int32),
            pltpu.VMEM((K + 16,), jnp.int32),
            pltpu.VMEM((K, D), jnp.float32),
            pltpu.VMEM((K, D), jnp.float32),
            pltpu.VMEM((BAND * D,), jnp.float32),
            pltpu.SemaphoreType.DMA,
            pltpu.SemaphoreType.DMA,
            pltpu.SemaphoreType.DMA,
            pltpu.SemaphoreType.DMA,
        ],
    )(h, src_s, dst_s, bounds, zero_rows)


_INV_N = np.float32(1.0 / N)


def _fold8(acc):
    a = acc[:4] + acc[4:]
    b = a[:2] + a[2:]
    return b[:1] + b[1:]


def _tc_layer_body(h_ref, p_ref, wa_ref, ba_ref, wb_ref, bb_ref, g_ref,
                   be_ref, o_ref, hh_ref):
    hp = h_ref[...] + p_ref[pl.ds(0, N), :]
    z = jnp.maximum(lax.dot(hp, wa_ref[...]) + ba_ref[...], 0.0)
    hh_ref[...] = jnp.maximum(lax.dot(z, wb_ref[...]) + bb_ref[...], 0.0)

    def mstep(t, acc):
        return acc + hh_ref[pl.ds(t * 8, 8), :]

    m = _fold8(lax.fori_loop(0, N // 8, mstep, jnp.zeros((8, D), jnp.float32),
                             unroll=10)) * _INV_N

    def vstep(base):
        def step(t, acc):
            c = hh_ref[pl.ds(base + t * 8, 8), :] - m
            return acc + c * c
        return _fold8(lax.fori_loop(0, N // 16, step,
                                    jnp.zeros((8, D), jnp.float32),
                                    unroll=5))

    v = (vstep(0) + vstep(N // 2)) * _INV_N
    o_ref[...] = (g_ref[...] * (hh_ref[...] - m)
                  * lax.rsqrt(v + 1e-5) + be_ref[...])


@jax.jit
def _tc_layer(h, p, wa, ba, wb, bb, g, be):
    return pl.pallas_call(
        _tc_layer_body,
        out_shape=jax.ShapeDtypeStruct((N, D), jnp.float32),
        scratch_shapes=[pltpu.VMEM((N, D), jnp.float32)],
    )(h, p, wa, ba, wb, bb, g, be)


def kernel(x, edge_index,
           w1a, b1a, w1b, b1b, g1, be1,
           w2a, b2a, w2b, b2b, g2, be2,
           w3a, b3a, w3b, b3b, g3, be3,
           w4a, b4a, w4b, b4b, g4, be4,
           w5a, b5a, w5b, b5b, g5, be5):
    dst_s, src_s = lax.sort((edge_index[1], edge_index[0]), num_keys=1,
                            is_stable=True)
    bounds = jnp.searchsorted(
        dst_s, jnp.arange(NW + 1, dtype=jnp.int32) * TROWS,
        side="left").astype(jnp.int32)
    src_s = jnp.concatenate([src_s, jnp.zeros((K,), jnp.int32)])
    dst_s = jnp.concatenate([dst_s, jnp.full((K,), NP, jnp.int32)])
    bounds_p = jnp.zeros((48,), jnp.int32).at[:NW + 1].set(bounds)
    zero_rows = jnp.zeros((BAND * D,), jnp.float32)

    w5b_p = jnp.zeros((D, D), jnp.float32).at[:, :2].set(w5b)
    b5b_p = jnp.zeros((D,), jnp.float32).at[:2].set(b5b)
    g5_p = jnp.zeros((D,), jnp.float32).at[:2].set(g5)
    be5_p = jnp.zeros((D,), jnp.float32).at[:2].set(be5)

    layers = [
        (w1a, b1a, w1b, b1b, g1, be1),
        (w2a, b2a, w2b, b2b, g2, be2),
        (w3a, b3a, w3b, b3b, g3, be3),
        (w4a, b4a, w4b, b4b, g4, be4),
        (w5a, b5a, w5b_p, b5b_p, g5_p, be5_p),
    ]

    h = x
    for (wa, ba, wb, bb, g, be) in layers:
        p = _sc_agg(h, src_s, dst_s, bounds_p, zero_rows).reshape(NP, D)
        h = _tc_layer(h, p, wa, ba.reshape(1, D), wb, bb.reshape(1, D),
                      g.reshape(1, D), be.reshape(1, D))
    return h[:, :2]


# speedup vs baseline: 5.2066x; 1.0008x over previous
import jax
import jax.numpy as jnp
import numpy as np
from jax import lax
from jax.experimental import pallas as pl
from jax.experimental.pallas import tpu as pltpu
from jax.experimental.pallas import tpu_sc as plsc

N = 10000
D = 128
E = 320000
NC = 2
NS = 16
NW = NC * NS
TROWS = 320
NP = NW * TROWS
K = 128
NCHP = E // K + 1
EP = NCHP * K
BAND = TROWS + 8


def _sc_agg_body(h_hbm, src_hbm, dst_hbm, bounds_hbm, zero_hbm, out_hbm,
                 bounds_v, src_v0, dst_v0, src_v1, dst_v1, rows_v0, rows_v1,
                 band_v, semg0, semg1, semi0, semi1):
    t = lax.axis_index("s") * NC + lax.axis_index("c")
    pltpu.sync_copy(zero_hbm, band_v)
    pltpu.sync_copy(bounds_hbm, bounds_v)
    b_lo = bounds_v[pl.ds(t, 16)][0]
    b_hi = bounds_v[pl.ds(t + 1, 16)][0]
    c_lo = lax.div(b_lo, K)
    c_hi = lax.div(b_hi + (K - 1), K)
    npairs = lax.max((c_hi - c_lo + 1) // 2, 1)
    rowbase = t * TROWS

    zero16 = jnp.zeros((16,), jnp.float32)

    def flush(s_prev, acc):
        for q in range(8):
            plsc.addupdate(band_v.at[pl.ds(s_prev * D + q * 16, 16)], acc[q])

    def start_idx(c, src_v, dst_v, semi):
        off = jnp.minimum(c, NCHP - 1) * K
        pltpu.async_copy(src_hbm.at[pl.ds(off, K)], src_v, semi)
        pltpu.async_copy(dst_hbm.at[pl.ds(off, K)], dst_v.at[pl.ds(0, K)],
                         semi)

    def wait_idx(src_v, dst_v, semi):
        pltpu.make_async_copy(src_hbm.at[pl.ds(0, K)], src_v, semi).wait()
        pltpu.make_async_copy(dst_hbm.at[pl.ds(0, K)],
                              dst_v.at[pl.ds(0, K)], semi).wait()

    def start_gather(src_v, rows_v, semg):
        pltpu.async_copy(h_hbm.at[src_v], rows_v, semg)

    def wait_gather(src_v, rows_v, semg):
        pltpu.make_async_copy(h_hbm.at[src_v], rows_v, semg).wait()

    def process(dst_v, rows_v, st):
        def group_body(g, st2):
            s_prev, acc = st2
            base = g * 16
            dvec = dst_v[pl.ds(base, 16)] - rowbase
            for j in range(16):
                s = dvec[j]
                s = jnp.where(jnp.logical_and(s >= 0, s < TROWS), s, TROWS)
                cond = s != s_prev

                @pl.when(cond)
                def _(s_prev=s_prev, acc=acc):
                    flush(s_prev, acc)

                r = [rows_v[base + j, pl.ds(q * 16, 16)] for q in range(8)]
                acc = tuple(
                    jnp.where(cond, r[q], acc[q] + r[q]) for q in range(8))
                s_prev = s
            return (s_prev, acc)

        return lax.fori_loop(0, K // 16, group_body, st)

    start_idx(c_lo, src_v0, dst_v0, semi0)
    wait_idx(src_v0, dst_v0, semi0)
    start_gather(src_v0, rows_v0, semg0)
    start_idx(c_lo + 1, src_v1, dst_v1, semi1)

    def pair_body(pi, st):
        a = c_lo + 2 * pi
        wait_idx(src_v1, dst_v1, semi1)
        start_gather(src_v1, rows_v1, semg1)
        wait_gather(src_v0, rows_v0, semg0)
        st = process(dst_v0, rows_v0, st)
        start_idx(a + 2, src_v0, dst_v0, semi0)
        wait_gather(src_v1, rows_v1, semg1)
        st = process(dst_v1, rows_v1, st)
        wait_idx(src_v0, dst_v0, semi0)
        start_gather(src_v0, rows_v0, semg0)
        start_idx(a + 3, src_v1, dst_v1, semi1)
        return st

    st = lax.fori_loop(0, npairs, pair_body,
                       (jnp.int32(TROWS), (zero16,) * 8))
    wait_gather(src_v0, rows_v0, semg0)
    wait_idx(src_v1, dst_v1, semi1)
    flush(st[0], st[1])
    pltpu.sync_copy(band_v.at[pl.ds(0, TROWS * D)],
                    out_hbm.at[pl.ds(rowbase * D, TROWS * D)])


@jax.jit
def _sc_agg(h, src_s, dst_s, bounds, zero_rows):
    mesh = plsc.VectorSubcoreMesh(core_axis_name="c", subcore_axis_name="s")
    return pl.kernel(
        _sc_agg_body,
        out_type=jax.ShapeDtypeStruct((NP * D,), jnp.float32),
        mesh=mesh,
        scratch_types=[
            pltpu.VMEM((48,), jnp.int32),
            pltpu.VMEM((K,), jnp.int32),
            pltpu.VMEM((K + 16,), jnp.int32),
            pltpu.VMEM((K,), jnp.int32),
            pltpu.VMEM((K + 16,), jnp.int32),
            pltpu.VMEM((K, D), jnp.float32),
            pltpu.VMEM((K, D), jnp.float32),
            pltpu.VMEM((BAND * D,), jnp.float32),
            pltpu.SemaphoreType.DMA,
            pltpu.SemaphoreType.DMA,
            pltpu.SemaphoreType.DMA,
            pltpu.SemaphoreType.DMA,
        ],
    )(h, src_s, dst_s, bounds, zero_rows)


_INV_N = np.float32(1.0 / N)


def _fold8(acc):
    a = acc[:4] + acc[4:]
    b = a[:2] + a[2:]
    return b[:1] + b[1:]


def _tc_layer_body(h_ref, p_ref, wa_ref, ba_ref, wb_ref, bb_ref, g_ref,
                   be_ref, o_ref, hh_ref):
    hp = h_ref[...] + p_ref[pl.ds(0, N), :]
    z = jnp.maximum(lax.dot(hp, wa_ref[...]) + ba_ref[...], 0.0)
    hh_ref[...] = jnp.maximum(lax.dot(z, wb_ref[...]) + bb_ref[...], 0.0)

    def mstep(t, acc):
        return acc + hh_ref[pl.ds(t * 8, 8), :]

    m = _fold8(lax.fori_loop(0, N // 8, mstep, jnp.zeros((8, D), jnp.float32),
                             unroll=10)) * _INV_N

    def vstep(base):
        def step(t, acc):
            c = hh_ref[pl.ds(base + t * 8, 8), :] - m
            return acc + c * c
        return _fold8(lax.fori_loop(0, N // 16, step,
                                    jnp.zeros((8, D), jnp.float32),
                                    unroll=5))

    v = (vstep(0) + vstep(N // 2)) * _INV_N
    o_ref[...] = (g_ref[...] * (hh_ref[...] - m)
                  * lax.rsqrt(v + 1e-5) + be_ref[...])


@jax.jit
def _tc_layer(h, p, wa, ba, wb, bb, g, be):
    return pl.pallas_call(
        _tc_layer_body,
        out_shape=jax.ShapeDtypeStruct((N, D), jnp.float32),
        scratch_shapes=[pltpu.VMEM((N, D), jnp.float32)],
    )(h, p, wa, ba, wb, bb, g, be)


def kernel(x, edge_index,
           w1a, b1a, w1b, b1b, g1, be1,
           w2a, b2a, w2b, b2b, g2, be2,
           w3a, b3a, w3b, b3b, g3, be3,
           w4a, b4a, w4b, b4b, g4, be4,
           w5a, b5a, w5b, b5b, g5, be5):
    dst_s, src_s = lax.sort((edge_index[1], edge_index[0]), num_keys=1,
                            is_stable=True)
    bounds = jnp.searchsorted(
        dst_s, jnp.arange(NW + 1, dtype=jnp.int32) * TROWS,
        side="left").astype(jnp.int32)
    src_s = jnp.concatenate([src_s, jnp.zeros((K,), jnp.int32)])
    dst_s = jnp.concatenate([dst_s, jnp.full((K,), NP, jnp.int32)])
    bounds_p = jnp.zeros((48,), jnp.int32).at[:NW + 1].set(bounds)
    zero_rows = jnp.zeros((BAND * D,), jnp.float32)

    w5b_p = jnp.zeros((D, D), jnp.float32).at[:, :2].set(w5b)
    b5b_p = jnp.zeros((D,), jnp.float32).at[:2].set(b5b)
    g5_p = jnp.zeros((D,), jnp.float32).at[:2].set(g5)
    be5_p = jnp.zeros((D,), jnp.float32).at[:2].set(be5)

    layers = [
        (w1a, b1a, w1b, b1b, g1, be1),
        (w2a, b2a, w2b, b2b, g2, be2),
        (w3a, b3a, w3b, b3b, g3, be3),
        (w4a, b4a, w4b, b4b, g4, be4),
        (w5a, b5a, w5b_p, b5b_p, g5_p, be5_p),
    ]

    h = x
    for (wa, ba, wb, bb, g, be) in layers:
        p = _sc_agg(h, src_s, dst_s, bounds_p, zero_rows).reshape(NP, D)
        h = _tc_layer(h, p, wa, ba.reshape(1, D), wb, bb.reshape(1, D),
                      g.reshape(1, D), be.reshape(1, D))
    return h[:, :2]
